# baseline (speedup 1.0000x reference)
import functools

import jax
import jax.numpy as jnp
from jax import lax
from jax.experimental import pallas as pl
from jax.experimental.pallas import tpu as pltpu
from jax.experimental.pallas import tpu_sc as plsc

N = 10000
D = 128
H = 128
C = 40
E = 320000
EPS = 1e-5

NC = 2
NS = 16
NW = NC * NS
EPW = E // NW
K = 128
BCH = 16
NBLK = 5
EPT = NBLK * BCH * K
EPAD = NW * EPT
DK = 80
DCHUNKS = EPW // DK
SROWS = 632
NP = SROWS * NS
TRASH = 10100


def _make_agg(width, interpret=False):
  mesh = plsc.VectorSubcoreMesh(core_axis_name="c", subcore_axis_name="s")

  @functools.partial(
      pl.kernel,
      out_type=jax.ShapeDtypeStruct((NC, NP, width), jnp.float32),
      mesh=mesh,
      interpret=interpret,
      scratch_types=[
          pltpu.VMEM((2, BCH, K), jnp.int32),
          pltpu.VMEM((2, BCH, K), jnp.int32),
          pltpu.VMEM((2, K, width), jnp.float32),
          pltpu.VMEM_SHARED((NP, width), jnp.float32),
          pltpu.SemaphoreType.DMA,
          pltpu.SemaphoreType.DMA,
      ],
  )
  def agg(y_hbm, src_hbm, dst_hbm, zeros_hbm, out_hbm, sidx, didx, rows, acc,
          gsem, isem):
    cid = lax.axis_index("c")
    sid = lax.axis_index("s")
    wid = sid * NC + cid
    pltpu.sync_copy(zeros_hbm.at[pl.ds(sid * SROWS, SROWS)],
                    acc.at[pl.ds(sid * SROWS, SROWS)])
    pltpu.sync_copy(src_hbm.at[wid, 0], sidx.at[0])
    pltpu.sync_copy(dst_hbm.at[wid, 0], didx.at[0])
    plsc.subcore_barrier()

    pltpu.async_copy(y_hbm.at[sidx.at[0, 0]], rows.at[0], gsem)

    def block(blk, carry):
      ib = lax.rem(blk, 2)
      nib = 1 - ib

      @pl.when(blk < NBLK - 1)
      def _():
        pltpu.async_copy(src_hbm.at[wid, blk + 1], sidx.at[nib], isem)
        pltpu.async_copy(dst_hbm.at[wid, blk + 1], didx.at[nib], isem)

      for ch in range(BCH):
        b = ch % 2
        pltpu.make_async_copy(y_hbm.at[sidx.at[ib, ch]], rows.at[b],
                              gsem).wait()
        if ch < BCH - 1:
          pltpu.async_copy(y_hbm.at[sidx.at[ib, ch + 1]], rows.at[1 - b],
                           gsem)
        else:
          @pl.when(blk < NBLK - 1)
          def _():
            pltpu.make_async_copy(src_hbm.at[wid, 0], sidx.at[nib],
                                  isem).wait()
            pltpu.make_async_copy(dst_hbm.at[wid, 0], didx.at[nib],
                                  isem).wait()
            pltpu.async_copy(y_hbm.at[sidx.at[nib, 0]], rows.at[1 - b], gsem)
        pltpu.sync_copy(rows.at[b], acc.at[didx.at[ib, ch]], add=True)
      return carry

    lax.fori_loop(0, NBLK, block, 0)
    plsc.subcore_barrier()
    pltpu.sync_copy(acc.at[pl.ds(sid * SROWS, SROWS)],
                    out_hbm.at[cid, pl.ds(sid * SROWS, SROWS)])

  return agg


_agg_h = _make_agg(H)


def _make_deg(interpret=False):
  mesh = plsc.VectorSubcoreMesh(core_axis_name="c", subcore_axis_name="s")

  @functools.partial(
      pl.kernel,
      out_type=jax.ShapeDtypeStruct((NW, NP), jnp.float32),
      mesh=mesh,
      interpret=interpret,
      scratch_types=[
          pltpu.VMEM((DCHUNKS, DK), jnp.int32),
          pltpu.VMEM((NP,), jnp.float32),
      ],
  )
  def deg_kernel(dst_hbm, zeros_hbm, out_hbm, didx, hist):
    cid = lax.axis_index("c")
    sid = lax.axis_index("s")
    wid = cid * NS + sid
    pltpu.sync_copy(zeros_hbm, hist)
    pltpu.sync_copy(dst_hbm.at[wid], didx)
    lane = lax.iota(jnp.int32, 16)

    def chunk(g, carry):
      for j in range(DK // 16):
        dv = didx[g, pl.ds(j * 16, 16)]
        for jj in range(16):
          d = dv[jj]
          b = jnp.bitwise_and(d, jnp.int32(-8))
          off = d - b
          v = hist[pl.ds(b, 16)]
          hist[pl.ds(b, 16)] = v + jnp.where(lane == off, 1.0, 0.0)
      return carry

    lax.fori_loop(0, DCHUNKS, chunk, 0)
    pltpu.sync_copy(hist, out_hbm.at[wid])

  return deg_kernel


_deg_kernel = _make_deg()


BLK = 1000


def _tc_degcol_fn(dh_ref, ones_ref, o_ref):
  o_ref[...] = lax.dot_general(dh_ref[...], ones_ref[...],
                               (((0,), (0,)), ((), ())),
                               preferred_element_type=jnp.float32)


def _dinv(d_ref):
  deg = d_ref[:, 0:1] + 1.0
  return lax.rsqrt(deg)


def _tc_scale_mm_fn(x_ref, w_ref, d_ref, o_ref):
  o_ref[...] = jnp.dot(x_ref[...], w_ref[...],
                       preferred_element_type=jnp.float32) * _dinv(d_ref)


def _tc_stats_fn(p0_ref, p1_ref, y_ref, b_ref, d_ref, t_ref, s_ref, q_ref):
  t = _dinv(d_ref) * (p0_ref[0] + p1_ref[0] + y_ref[...]) + b_ref[...]
  t_ref[...] = t

  @pl.when(pl.program_id(0) == 0)
  def _():
    s_ref[...] = jnp.zeros_like(s_ref)
    q_ref[...] = jnp.zeros_like(q_ref)

  s_ref[...] += jnp.sum(t, axis=0, keepdims=True)
  q_ref[...] += jnp.sum(t * t, axis=0, keepdims=True)


def _tc_norm_mm_fn(t_ref, s_ref, q_ref, g_ref, be_ref, w_ref, d_ref, o_ref):
  mean = s_ref[...] * (1.0 / N)
  var = q_ref[...] * (1.0 / N) - mean * mean
  rstd = lax.rsqrt(var + EPS)
  h = jnp.maximum((t_ref[...] - mean) * rstd * g_ref[...] + be_ref[...], 0.0)
  o_ref[...] = jnp.dot(h, w_ref[...],
                       preferred_element_type=jnp.float32) * _dinv(d_ref)


def _tc_final_fn(p0_ref, p1_ref, y_ref, b_ref, d_ref, o_ref):
  t = _dinv(d_ref) * (p0_ref[0] + p1_ref[0] + y_ref[...]) + b_ref[...]
  col = lax.broadcasted_iota(jnp.int32, t.shape, 1)
  t = jnp.where(col < C, t, -1e30)
  m = jnp.max(t, axis=1, keepdims=True)
  lse = jnp.log(jnp.sum(jnp.exp(t - m), axis=1, keepdims=True)) + m
  o_ref[...] = (t - lse)[:, :C]


def _row_spec(w):
  return pl.BlockSpec((BLK, w), lambda i: (i, 0))


def _p_specs(w):
  return [pl.BlockSpec((1, BLK, w), lambda i: (0, i, 0)),
          pl.BlockSpec((1, BLK, w), lambda i: (1, i, 0))]


def _full_spec(r, c):
  return pl.BlockSpec((r, c), lambda i: (0, 0))


def kernel(x, edge_index, W1, b1, g1, be1, W2, b2, g2, be2, W3, b3):
  f32 = jnp.float32
  npad = EPAD - E
  src3 = jnp.concatenate(
      [edge_index[0], jnp.zeros((npad,), edge_index.dtype)]).reshape(
          NW, NBLK, BCH, K)
  trash = N + jnp.arange(npad, dtype=edge_index.dtype) % (NP - N)
  dst3 = jnp.concatenate([edge_index[1], trash]).reshape(NW, NBLK, BCH, K)
  dst3w = edge_index[1].reshape(NW, DCHUNKS, DK)
  zeros_h = jnp.zeros((NP, H), f32)
  zeros_n = jnp.zeros((NP,), f32)
  ones_w = jnp.ones((NW, H), f32)
  W3p = jnp.pad(W3, ((0, 0), (0, H - C)))
  b3p = jnp.pad(b3, (0, H - C)).reshape(1, H)
  b1r = b1.reshape(1, H)
  b2r = b2.reshape(1, H)
  g1r, be1r = g1.reshape(1, H), be1.reshape(1, H)
  g2r, be2r = g2.reshape(1, H), be2.reshape(1, H)

  degh = _deg_kernel(dst3w, zeros_n)

  dcol = pl.pallas_call(
      _tc_degcol_fn,
      grid=(1,),
      in_specs=[pl.BlockSpec((NW, NP), lambda i: (0, 0)),
                pl.BlockSpec((NW, H), lambda i: (0, 0))],
      out_specs=pl.BlockSpec((NP, H), lambda i: (0, 0)),
      out_shape=jax.ShapeDtypeStruct((NP, H), f32),
  )(degh, ones_w)

  grid = (N // BLK,)
  degs = [_row_spec(H)]

  y1 = pl.pallas_call(
      _tc_scale_mm_fn,
      grid=grid,
      in_specs=[_row_spec(D), _full_spec(D, H)] + degs,
      out_specs=_row_spec(H),
      out_shape=jax.ShapeDtypeStruct((N, H), f32),
  )(x, W1, dcol)

  def stats(p, y, br):
    return pl.pallas_call(
        _tc_stats_fn,
        grid=grid,
        in_specs=_p_specs(H) + [_row_spec(H), _full_spec(1, H)] + degs,
        out_specs=[_row_spec(H), _full_spec(1, H), _full_spec(1, H)],
        out_shape=[jax.ShapeDtypeStruct((N, H), f32),
                   jax.ShapeDtypeStruct((1, H), f32),
                   jax.ShapeDtypeStruct((1, H), f32)],
    )(p, p, y, br, dcol)

  def norm_mm(t, s, q, gr, ber, w):
    return pl.pallas_call(
        _tc_norm_mm_fn,
        grid=grid,
        in_specs=[_row_spec(H), _full_spec(1, H), _full_spec(1, H),
                  _full_spec(1, H), _full_spec(1, H), _full_spec(H, H)]
        + degs,
        out_specs=_row_spec(H),
        out_shape=jax.ShapeDtypeStruct((N, H), f32),
    )(t, s, q, gr, ber, w, dcol)

  p1 = _agg_h(y1, src3, dst3, zeros_h)
  t1, s1, q1 = stats(p1, y1, b1r)
  y2 = norm_mm(t1, s1, q1, g1r, be1r, W2)

  p2 = _agg_h(y2, src3, dst3, zeros_h)
  t2, s2, q2 = stats(p2, y2, b2r)
  y3 = norm_mm(t2, s2, q2, g2r, be2r, W3p)

  p3 = _agg_h(y3, src3, dst3, zeros_h)

  out = pl.pallas_call(
      _tc_final_fn,
      grid=grid,
      in_specs=_p_specs(H) + [_row_spec(H), _full_spec(1, H)] + degs,
      out_specs=_row_spec(C),
      out_shape=jax.ShapeDtypeStruct((N, C), f32),
  )(p3, p3, y3, b3p, dcol)
  return out


# speedup vs baseline: 1.2862x; 1.2862x over previous
import functools

import jax
import jax.numpy as jnp
from jax import lax
from jax.experimental import pallas as pl
from jax.experimental.pallas import tpu as pltpu
from jax.experimental.pallas import tpu_sc as plsc

N = 10000
D = 128
H = 128
C = 40
E = 320000
EPS = 1e-5

NC = 2
NS = 16
NW = NC * NS
EPW = E // NW
K = 128
CHUNKS = 157
EPT = CHUNKS * K
EPAD = NS * EPT
DK = 80
DCHUNKS = EPW // DK
SROWS = 632
NP = SROWS * NS
DEGW = 16
HALF = N // 2
NLOC = 5120
LROWS = NLOC // NS
TRASH = 5112


def _make_agg(width, interpret=False):
  mesh = plsc.VectorSubcoreMesh(core_axis_name="c", subcore_axis_name="s")

  @functools.partial(
      pl.kernel,
      out_type=jax.ShapeDtypeStruct((NC, NLOC, width), jnp.float32),
      mesh=mesh,
      interpret=interpret,
      scratch_types=[
          pltpu.VMEM((CHUNKS, K), jnp.int32),
          pltpu.VMEM((CHUNKS, K), jnp.int32),
          pltpu.VMEM((2, K, width), jnp.float32),
          pltpu.VMEM_SHARED((NLOC, width), jnp.float32),
          pltpu.SemaphoreType.DMA,
      ],
  )
  def agg(y_hbm, src_hbm, dst_hbm, zeros_hbm, out_hbm, sidx, didx, rows, acc,
          sem):
    cid = lax.axis_index("c")
    sid = lax.axis_index("s")
    pltpu.sync_copy(zeros_hbm.at[pl.ds(sid * LROWS, LROWS)],
                    acc.at[pl.ds(sid * LROWS, LROWS)])
    pltpu.sync_copy(src_hbm.at[sid], sidx)
    pltpu.sync_copy(dst_hbm.at[sid], didx)

    base = cid * HALF

    def xform(g, carry):
      for j in range(K // 16):
        v = didx[g, pl.ds(j * 16, 16)]
        loc = v - base
        ok = (loc >= 0) & (loc < HALF)
        trash = HALF + jnp.bitwise_and(v, 63)
        didx[g, pl.ds(j * 16, 16)] = jnp.where(ok, loc, trash)
      return carry

    lax.fori_loop(0, CHUNKS, xform, 0)
    plsc.subcore_barrier()

    pltpu.async_copy(y_hbm.at[sidx.at[0]], rows.at[0], sem)

    def step(g, b):
      pltpu.make_async_copy(y_hbm.at[sidx.at[g]], rows.at[b], sem).wait()

      @pl.when(g + 1 < CHUNKS)
      def _():
        pltpu.async_copy(y_hbm.at[sidx.at[g + 1]], rows.at[1 - b], sem)

      pltpu.sync_copy(rows.at[b], acc.at[didx.at[g]], add=True)

    def outer(go, carry):
      step(go * 2, 0)
      step(go * 2 + 1, 1)
      return carry

    lax.fori_loop(0, CHUNKS // 2, outer, 0)
    step(CHUNKS - 1, 0)
    plsc.subcore_barrier()
    pltpu.sync_copy(acc.at[pl.ds(sid * LROWS, LROWS)],
                    out_hbm.at[cid, pl.ds(sid * LROWS, LROWS)])

  return agg


_agg_h = _make_agg(H)


def _make_deg(interpret=False):
  mesh = plsc.VectorSubcoreMesh(core_axis_name="c", subcore_axis_name="s")

  @functools.partial(
      pl.kernel,
      out_type=jax.ShapeDtypeStruct((NW, NP), jnp.float32),
      mesh=mesh,
      interpret=interpret,
      scratch_types=[
          pltpu.VMEM((DCHUNKS, DK), jnp.int32),
          pltpu.VMEM((NP,), jnp.float32),
      ],
  )
  def deg_kernel(dst_hbm, zeros_hbm, out_hbm, didx, hist):
    cid = lax.axis_index("c")
    sid = lax.axis_index("s")
    wid = cid * NS + sid
    pltpu.sync_copy(zeros_hbm, hist)
    pltpu.sync_copy(dst_hbm.at[wid], didx)
    lane = lax.iota(jnp.int32, 16)

    def chunk(g, carry):
      for j in range(DK // 16):
        dv = didx[g, pl.ds(j * 16, 16)]
        for jj in range(16):
          d = dv[jj]
          b = jnp.bitwise_and(d, jnp.int32(-8))
          off = d - b
          v = hist[pl.ds(b, 16)]
          hist[pl.ds(b, 16)] = v + jnp.where(lane == off, 1.0, 0.0)
      return carry

    lax.fori_loop(0, DCHUNKS, chunk, 0)
    pltpu.sync_copy(hist, out_hbm.at[wid])

  return deg_kernel


_deg_kernel = _make_deg()


BLK = 1000


def _tc_degcol_fn(dh_ref, ones_ref, o_ref):
  o_ref[...] = lax.dot_general(dh_ref[...], ones_ref[...],
                               (((0,), (0,)), ((), ())),
                               preferred_element_type=jnp.float32)


def _dinv(d_ref):
  deg = d_ref[:, 0:1] + 1.0
  return lax.rsqrt(deg)


def _tc_scale_mm_fn(x_ref, w_ref, d_ref, o_ref):
  o_ref[...] = jnp.dot(x_ref[...], w_ref[...],
                       preferred_element_type=jnp.float32) * _dinv(d_ref)


def _tc_stats_fn(p_ref, y_ref, b_ref, d_ref, t_ref, s_ref, q_ref):
  t = _dinv(d_ref) * (p_ref[0] + y_ref[...]) + b_ref[...]
  t_ref[...] = t

  @pl.when(pl.program_id(0) == 0)
  def _():
    s_ref[...] = jnp.zeros_like(s_ref)
    q_ref[...] = jnp.zeros_like(q_ref)

  s_ref[...] += jnp.sum(t, axis=0, keepdims=True)
  q_ref[...] += jnp.sum(t * t, axis=0, keepdims=True)


def _tc_norm_mm_fn(t_ref, s_ref, q_ref, g_ref, be_ref, w_ref, d_ref, o_ref):
  mean = s_ref[...] * (1.0 / N)
  var = q_ref[...] * (1.0 / N) - mean * mean
  rstd = lax.rsqrt(var + EPS)
  h = jnp.maximum((t_ref[...] - mean) * rstd * g_ref[...] + be_ref[...], 0.0)
  o_ref[...] = jnp.dot(h, w_ref[...],
                       preferred_element_type=jnp.float32) * _dinv(d_ref)


def _tc_final_fn(p_ref, y_ref, b_ref, d_ref, o_ref):
  t = _dinv(d_ref) * (p_ref[0] + y_ref[...]) + b_ref[...]
  col = lax.broadcasted_iota(jnp.int32, t.shape, 1)
  t = jnp.where(col < C, t, -1e30)
  m = jnp.max(t, axis=1, keepdims=True)
  lse = jnp.log(jnp.sum(jnp.exp(t - m), axis=1, keepdims=True)) + m
  o_ref[...] = (t - lse)[:, :C]


def _row_spec(w):
  return pl.BlockSpec((BLK, w), lambda i: (i, 0))


def _p_spec(w):
  return pl.BlockSpec((1, BLK, w), lambda i: (i // 5, i % 5, 0))


def _full_spec(r, c):
  return pl.BlockSpec((r, c), lambda i: (0, 0))


def kernel(x, edge_index, W1, b1, g1, be1, W2, b2, g2, be2, W3, b3):
  f32 = jnp.float32
  npad = EPAD - E
  src3 = jnp.concatenate(
      [edge_index[0], jnp.zeros((npad,), edge_index.dtype)]).reshape(
          NS, CHUNKS, K)
  dpad = N + jnp.arange(npad, dtype=edge_index.dtype) % 64
  dst3 = jnp.concatenate([edge_index[1], dpad]).reshape(NS, CHUNKS, K)
  dst3w = edge_index[1].reshape(NW, DCHUNKS, DK)
  zeros_h = jnp.zeros((NLOC, H), f32)
  zeros_n = jnp.zeros((NP,), f32)
  ones_w = jnp.ones((NW, H), f32)
  W3p = jnp.pad(W3, ((0, 0), (0, H - C)))
  b3p = jnp.pad(b3, (0, H - C)).reshape(1, H)
  b1r = b1.reshape(1, H)
  b2r = b2.reshape(1, H)
  g1r, be1r = g1.reshape(1, H), be1.reshape(1, H)
  g2r, be2r = g2.reshape(1, H), be2.reshape(1, H)

  degh = _deg_kernel(dst3w, zeros_n)

  dcol = pl.pallas_call(
      _tc_degcol_fn,
      grid=(1,),
      in_specs=[pl.BlockSpec((NW, NP), lambda i: (0, 0)),
                pl.BlockSpec((NW, H), lambda i: (0, 0))],
      out_specs=pl.BlockSpec((NP, H), lambda i: (0, 0)),
      out_shape=jax.ShapeDtypeStruct((NP, H), f32),
  )(degh, ones_w)

  grid = (N // BLK,)
  degs = [_row_spec(H)]

  y1 = pl.pallas_call(
      _tc_scale_mm_fn,
      grid=grid,
      in_specs=[_row_spec(D), _full_spec(D, H)] + degs,
      out_specs=_row_spec(H),
      out_shape=jax.ShapeDtypeStruct((N, H), f32),
  )(x, W1, dcol)

  def stats(p, y, br):
    return pl.pallas_call(
        _tc_stats_fn,
        grid=grid,
        in_specs=[_p_spec(H), _row_spec(H), _full_spec(1, H)] + degs,
        out_specs=[_row_spec(H), _full_spec(1, H), _full_spec(1, H)],
        out_shape=[jax.ShapeDtypeStruct((N, H), f32),
                   jax.ShapeDtypeStruct((1, H), f32),
                   jax.ShapeDtypeStruct((1, H), f32)],
    )(p, y, br, dcol)

  def norm_mm(t, s, q, gr, ber, w):
    return pl.pallas_call(
        _tc_norm_mm_fn,
        grid=grid,
        in_specs=[_row_spec(H), _full_spec(1, H), _full_spec(1, H),
                  _full_spec(1, H), _full_spec(1, H), _full_spec(H, H)]
        + degs,
        out_specs=_row_spec(H),
        out_shape=jax.ShapeDtypeStruct((N, H), f32),
    )(t, s, q, gr, ber, w, dcol)

  p1 = _agg_h(y1, src3, dst3, zeros_h)
  t1, s1, q1 = stats(p1, y1, b1r)
  y2 = norm_mm(t1, s1, q1, g1r, be1r, W2)

  p2 = _agg_h(y2, src3, dst3, zeros_h)
  t2, s2, q2 = stats(p2, y2, b2r)
  y3 = norm_mm(t2, s2, q2, g2r, be2r, W3p)

  p3 = _agg_h(y3, src3, dst3, zeros_h)

  out = pl.pallas_call(
      _tc_final_fn,
      grid=grid,
      in_specs=[_p_spec(H), _row_spec(H), _full_spec(1, H)] + degs,
      out_specs=_row_spec(C),
      out_shape=jax.ShapeDtypeStruct((N, C), f32),
  )(p3, y3, b3p, dcol)
  return out


# speedup vs baseline: 1.4478x; 1.1256x over previous
import functools

import jax
import jax.numpy as jnp
from jax import lax
from jax.experimental import pallas as pl
from jax.experimental.pallas import tpu as pltpu
from jax.experimental.pallas import tpu_sc as plsc

N = 10000
D = 128
H = 128
C = 40
E = 320000
EPS = 1e-5

NC = 2
NS = 16
NW = NC * NS
EPW = E // NW
K = 80
CHUNKS = 250
EPT = CHUNKS * K
EPAD = NS * EPT
DK = 80
DCHUNKS = EPW // DK
SROWS = 632
NP = SROWS * NS
DEGW = 16
HALF = N // 2
NLOC = 5120
LROWS = NLOC // NS
TRASH = 5112


def _make_agg(width, interpret=False):
  mesh = plsc.VectorSubcoreMesh(core_axis_name="c", subcore_axis_name="s")

  @functools.partial(
      pl.kernel,
      out_type=jax.ShapeDtypeStruct((NC, NLOC, width), jnp.float32),
      mesh=mesh,
      interpret=interpret,
      scratch_types=[
          pltpu.VMEM((CHUNKS, K), jnp.int32),
          pltpu.VMEM((CHUNKS, K), jnp.int32),
          pltpu.VMEM((2, K, width), jnp.float32),
          pltpu.VMEM_SHARED((NLOC, width), jnp.float32),
          pltpu.SemaphoreType.DMA,
      ],
  )
  def agg(y_hbm, src_hbm, dst_hbm, zeros_hbm, out_hbm, sidx, didx, rows, acc,
          sem):
    cid = lax.axis_index("c")
    sid = lax.axis_index("s")
    pltpu.sync_copy(zeros_hbm.at[pl.ds(sid * LROWS, LROWS)],
                    acc.at[pl.ds(sid * LROWS, LROWS)])
    pltpu.sync_copy(src_hbm.at[sid], sidx)
    pltpu.sync_copy(dst_hbm.at[sid], didx)

    base = cid * HALF

    def xform(g, carry):
      for j in range(K // 16):
        v = didx[g, pl.ds(j * 16, 16)]
        loc = v - base
        ok = (loc >= 0) & (loc < HALF)
        trash = HALF + jnp.bitwise_and(v, 63)
        didx[g, pl.ds(j * 16, 16)] = jnp.where(ok, loc, trash)
      return carry

    lax.fori_loop(0, CHUNKS, xform, 0)
    plsc.subcore_barrier()

    pltpu.async_copy(y_hbm.at[sidx.at[0]], rows.at[0], sem)

    def step(g, b):
      pltpu.make_async_copy(y_hbm.at[sidx.at[g]], rows.at[b], sem).wait()

      @pl.when(g + 1 < CHUNKS)
      def _():
        pltpu.async_copy(y_hbm.at[sidx.at[g + 1]], rows.at[1 - b], sem)

      pltpu.sync_copy(rows.at[b], acc.at[didx.at[g]], add=True)

    def outer(go, carry):
      step(go * 2, 0)
      step(go * 2 + 1, 1)
      return carry

    lax.fori_loop(0, CHUNKS // 2, outer, 0)
    plsc.subcore_barrier()
    pltpu.sync_copy(acc.at[pl.ds(sid * LROWS, LROWS)],
                    out_hbm.at[cid, pl.ds(sid * LROWS, LROWS)])

  return agg


_agg_h = _make_agg(H)


def _make_deg(interpret=False):
  mesh = plsc.VectorSubcoreMesh(core_axis_name="c", subcore_axis_name="s")

  @functools.partial(
      pl.kernel,
      out_type=jax.ShapeDtypeStruct((NW, NP), jnp.float32),
      mesh=mesh,
      interpret=interpret,
      scratch_types=[
          pltpu.VMEM((DCHUNKS, DK), jnp.int32),
          pltpu.VMEM((NP,), jnp.float32),
      ],
  )
  def deg_kernel(dst_hbm, zeros_hbm, out_hbm, didx, hist):
    cid = lax.axis_index("c")
    sid = lax.axis_index("s")
    wid = cid * NS + sid
    pltpu.sync_copy(zeros_hbm, hist)
    pltpu.sync_copy(dst_hbm.at[wid], didx)
    lane = lax.iota(jnp.int32, 16)

    def chunk(g, carry):
      for j in range(DK // 16):
        dv = didx[g, pl.ds(j * 16, 16)]
        for jj in range(16):
          d = dv[jj]
          b = jnp.bitwise_and(d, jnp.int32(-8))
          off = d - b
          v = hist[pl.ds(b, 16)]
          hist[pl.ds(b, 16)] = v + jnp.where(lane == off, 1.0, 0.0)
      return carry

    lax.fori_loop(0, DCHUNKS, chunk, 0)
    pltpu.sync_copy(hist, out_hbm.at[wid])

  return deg_kernel


_deg_kernel = _make_deg()


BLK = 1000


def _tc_degcol_fn(dh_ref, ones_ref, o_ref):
  o_ref[...] = lax.dot_general(dh_ref[...], ones_ref[...],
                               (((0,), (0,)), ((), ())),
                               preferred_element_type=jnp.float32)


def _dinv(d_ref):
  deg = d_ref[:, 0:1] + 1.0
  return lax.rsqrt(deg)


def _tc_scale_mm_fn(x_ref, w_ref, d_ref, o_ref):
  o_ref[...] = jnp.dot(x_ref[...], w_ref[...],
                       preferred_element_type=jnp.float32) * _dinv(d_ref)


def _tc_stats_fn(p_ref, y_ref, b_ref, d_ref, t_ref, s_ref, q_ref):
  t = _dinv(d_ref) * (p_ref[0] + y_ref[...]) + b_ref[...]
  t_ref[...] = t

  @pl.when(pl.program_id(0) == 0)
  def _():
    s_ref[...] = jnp.zeros_like(s_ref)
    q_ref[...] = jnp.zeros_like(q_ref)

  s_ref[...] += jnp.sum(t, axis=0, keepdims=True)
  q_ref[...] += jnp.sum(t * t, axis=0, keepdims=True)


def _tc_norm_mm_fn(t_ref, s_ref, q_ref, g_ref, be_ref, w_ref, d_ref, o_ref):
  mean = s_ref[...] * (1.0 / N)
  var = q_ref[...] * (1.0 / N) - mean * mean
  rstd = lax.rsqrt(var + EPS)
  h = jnp.maximum((t_ref[...] - mean) * rstd * g_ref[...] + be_ref[...], 0.0)
  o_ref[...] = jnp.dot(h, w_ref[...],
                       preferred_element_type=jnp.float32) * _dinv(d_ref)


def _tc_final_fn(p_ref, y_ref, b_ref, d_ref, o_ref):
  t = _dinv(d_ref) * (p_ref[0] + y_ref[...]) + b_ref[...]
  col = lax.broadcasted_iota(jnp.int32, t.shape, 1)
  t = jnp.where(col < C, t, -1e30)
  m = jnp.max(t, axis=1, keepdims=True)
  lse = jnp.log(jnp.sum(jnp.exp(t - m), axis=1, keepdims=True)) + m
  o_ref[...] = (t - lse)[:, :C]


def _row_spec(w):
  return pl.BlockSpec((BLK, w), lambda i: (i, 0))


def _p_spec(w):
  return pl.BlockSpec((1, BLK, w), lambda i: (i // 5, i % 5, 0))


def _full_spec(r, c):
  return pl.BlockSpec((r, c), lambda i: (0, 0))


def kernel(x, edge_index, W1, b1, g1, be1, W2, b2, g2, be2, W3, b3):
  f32 = jnp.float32
  npad = EPAD - E
  src3 = jnp.concatenate(
      [edge_index[0], jnp.zeros((npad,), edge_index.dtype)]).reshape(
          NS, CHUNKS, K)
  dpad = N + jnp.arange(npad, dtype=edge_index.dtype) % 64
  dst3 = jnp.concatenate([edge_index[1], dpad]).reshape(NS, CHUNKS, K)
  dst3w = edge_index[1].reshape(NW, DCHUNKS, DK)
  zeros_h = jnp.zeros((NLOC, H), f32)
  zeros_n = jnp.zeros((NP,), f32)
  ones_w = jnp.ones((NW, H), f32)
  W3p = jnp.pad(W3, ((0, 0), (0, H - C)))
  b3p = jnp.pad(b3, (0, H - C)).reshape(1, H)
  b1r = b1.reshape(1, H)
  b2r = b2.reshape(1, H)
  g1r, be1r = g1.reshape(1, H), be1.reshape(1, H)
  g2r, be2r = g2.reshape(1, H), be2.reshape(1, H)

  degh = _deg_kernel(dst3w, zeros_n)

  dcol = pl.pallas_call(
      _tc_degcol_fn,
      grid=(1,),
      in_specs=[pl.BlockSpec((NW, NP), lambda i: (0, 0)),
                pl.BlockSpec((NW, H), lambda i: (0, 0))],
      out_specs=pl.BlockSpec((NP, H), lambda i: (0, 0)),
      out_shape=jax.ShapeDtypeStruct((NP, H), f32),
  )(degh, ones_w)

  grid = (N // BLK,)
  degs = [_row_spec(H)]

  y1 = pl.pallas_call(
      _tc_scale_mm_fn,
      grid=grid,
      in_specs=[_row_spec(D), _full_spec(D, H)] + degs,
      out_specs=_row_spec(H),
      out_shape=jax.ShapeDtypeStruct((N, H), f32),
  )(x, W1, dcol)

  def stats(p, y, br):
    return pl.pallas_call(
        _tc_stats_fn,
        grid=grid,
        in_specs=[_p_spec(H), _row_spec(H), _full_spec(1, H)] + degs,
        out_specs=[_row_spec(H), _full_spec(1, H), _full_spec(1, H)],
        out_shape=[jax.ShapeDtypeStruct((N, H), f32),
                   jax.ShapeDtypeStruct((1, H), f32),
                   jax.ShapeDtypeStruct((1, H), f32)],
    )(p, y, br, dcol)

  def norm_mm(t, s, q, gr, ber, w):
    return pl.pallas_call(
        _tc_norm_mm_fn,
        grid=grid,
        in_specs=[_row_spec(H), _full_spec(1, H), _full_spec(1, H),
                  _full_spec(1, H), _full_spec(1, H), _full_spec(H, H)]
        + degs,
        out_specs=_row_spec(H),
        out_shape=jax.ShapeDtypeStruct((N, H), f32),
    )(t, s, q, gr, ber, w, dcol)

  p1 = _agg_h(y1, src3, dst3, zeros_h)
  t1, s1, q1 = stats(p1, y1, b1r)
  y2 = norm_mm(t1, s1, q1, g1r, be1r, W2)

  p2 = _agg_h(y2, src3, dst3, zeros_h)
  t2, s2, q2 = stats(p2, y2, b2r)
  y3 = norm_mm(t2, s2, q2, g2r, be2r, W3p)

  p3 = _agg_h(y3, src3, dst3, zeros_h)

  out = pl.pallas_call(
      _tc_final_fn,
      grid=grid,
      in_specs=[_p_spec(H), _row_spec(H), _full_spec(1, H)] + degs,
      out_specs=_row_spec(C),
      out_shape=jax.ShapeDtypeStruct((N, C), f32),
  )(p3, y3, b3p, dcol)
  return out
